# baseline (speedup 1.0000x reference)
import functools

import jax
import jax.numpy as jnp
from jax import lax
from jax.experimental import pallas as pl
from jax.experimental.pallas import tpu as pltpu
from jax.experimental.pallas import tpu_sc as plsc

N = 10000
E = 320000
D = 128

NC = 2
NS = 16
L = 16
NW = NC * NS
C = 128
DW = D // 2
NCHUNK = E // C
KMAX = -(-NCHUNK // NW)
KMAX4 = -(-KMAX // 4)
NPAD = 10240
RPT = NPAD // NS


def _h_body(x_ref, w_ref, b_ref, o_ref):
  h = jnp.dot(x_ref[...], w_ref[...], preferred_element_type=jnp.float32)
  o_ref[...] = (h + b_ref[...]).astype(jnp.bfloat16)


def _tc_h(x, w, b):
  return pl.pallas_call(
      _h_body,
      grid=(10,),
      in_specs=[
          pl.BlockSpec((N // 10, D), lambda i: (i, 0)),
          pl.BlockSpec((D, D), lambda i: (0, 0)),
          pl.BlockSpec((1, D), lambda i: (0, 0)),
      ],
      out_specs=pl.BlockSpec((N // 10, D), lambda i: (i, 0)),
      out_shape=jax.ShapeDtypeStruct((N, D), jnp.bfloat16),
  )(x, w, b.reshape(1, D))


def _merge_body(p_ref, d_ref, o_ref):
  dsum = d_ref[0, :] + d_ref[1, :] + 1e-16
  o_ref[...] = (p_ref[0] + p_ref[1]) * (1.0 / dsum)[:, None]


def _tc_merge(opart, dpart):
  return pl.pallas_call(
      _merge_body,
      grid=(NPAD // 1024,),
      in_specs=[
          pl.BlockSpec((2, 1024, D), lambda i: (0, i, 0)),
          pl.BlockSpec((2, 1024), lambda i: (0, i)),
      ],
      out_specs=pl.BlockSpec((1024, D), lambda i: (i, 0)),
      out_shape=jax.ShapeDtypeStruct((NPAD, D), jnp.float32),
  )(opart, dpart)[:N]


def _edge1_body(h_hbm, src_hbm, dst_hbm, att_hbm,
                ex_hbm, dpart_hbm,
                sidxs, didxs, is0, is1, is2, is3,
                hs0, hd0, sA0, sB0,
                hs1, hd1, sA1, sB1,
                lg, attv, zb, dshared):
  cid = lax.axis_index("c")
  sid = lax.axis_index("s")
  wid = cid * NS + sid
  r0 = sid * RPT
  isems = (is0, is1, is2, is3)

  def _z(i, _):
    zb[pl.ds(i * L, L)] = jnp.zeros((L,), jnp.float32)
    return 0
  lax.fori_loop(0, RPT // L, _z, 0)
  pltpu.sync_copy(zb, dshared.at[pl.ds(r0, RPT)])

  pltpu.sync_copy(att_hbm, attv)
  av = [plsc.bitcast(attv[pl.ds(r * L, L)], jnp.bfloat16)
        for r in range(DW // L)]
  plsc.subcore_barrier()

  gsets = ((hs0, hd0, sA0, sB0), (hs1, hd1, sA1, sB1))

  def istage(k, q):
    chunk = wid + k * NW

    @pl.when(chunk < NCHUNK)
    def _():
      off = chunk * C
      pltpu.async_copy(src_hbm.at[pl.ds(off, C)], sidxs.at[q], isems[q])
      pltpu.async_copy(dst_hbm.at[pl.ds(off, C)], didxs.at[q], isems[q])

  def gstage(k, q, p):
    hs, hd, sA, sB = gsets[p]
    chunk = wid + k * NW

    @pl.when(chunk < NCHUNK)
    def _():
      off = chunk * C
      pltpu.make_async_copy(
          src_hbm.at[pl.ds(off, C)], sidxs.at[q], isems[q]).wait()
      pltpu.make_async_copy(
          dst_hbm.at[pl.ds(off, C)], didxs.at[q], isems[q]).wait()
      pltpu.async_copy(h_hbm.at[sidxs.at[q]], hs, sA)
      pltpu.async_copy(h_hbm.at[didxs.at[q]], hd, sB)

  def cstage(k, q, p):
    hs, hd, sA, sB = gsets[p]
    chunk = wid + k * NW

    @pl.when(chunk < NCHUNK)
    def _():
      off = chunk * C
      pltpu.make_async_copy(h_hbm.at[sidxs.at[q]], hs, sA).wait()
      pltpu.make_async_copy(h_hbm.at[didxs.at[q]], hd, sB).wait()

      @plsc.parallel_loop(0, C, unroll=8)
      def _(e):
        acc = jnp.zeros((L,), jnp.float32)
        for r in range(DW // L):
          zs = plsc.bitcast(hs[e, pl.ds(r * L, L)], jnp.bfloat16)
          zd = plsc.bitcast(hd[e, pl.ds(r * L, L)], jnp.bfloat16)
          z = zs + zd
          t = jnp.maximum(z, z * jnp.bfloat16(0.2))
          pa, pb = plsc.unpack(t * av[r], format=plsc.PackFormat.INTERLEAVED)
          acc = acc + pa + pb
        lane = lax.iota(jnp.int32, L)
        for s in (8, 4, 2, 1):
          acc = acc + jnp.take_along_axis(
              acc, lane ^ s, axis=0, mode="promise_in_bounds")
        plsc.store_scatter(lg, [jnp.full((L,), e, jnp.int32)], acc)

      for g in range(C // L):
        v = lg[pl.ds(g * L, L)]
        lg[pl.ds(g * L, L)] = jnp.exp(jnp.minimum(v, 60.0))
      pltpu.sync_copy(lg, ex_hbm.at[pl.ds(off, C)])
      pltpu.sync_copy(lg, dshared.at[didxs.at[q]], add=True)

  istage(0, 0)
  istage(1, 1)
  istage(2, 2)
  gstage(0, 0, 0)
  gstage(1, 1, 1)

  def loop_body(j, _):
    k0 = 4 * j
    istage(k0 + 3, 3)
    cstage(k0, 0, 0)
    gstage(k0 + 2, 2, 0)
    istage(k0 + 4, 0)
    cstage(k0 + 1, 1, 1)
    gstage(k0 + 3, 3, 1)
    istage(k0 + 5, 1)
    cstage(k0 + 2, 2, 0)
    gstage(k0 + 4, 0, 0)
    istage(k0 + 6, 2)
    cstage(k0 + 3, 3, 1)
    gstage(k0 + 5, 1, 1)
    return 0

  lax.fori_loop(0, KMAX4, loop_body, 0)
  plsc.subcore_barrier()
  pltpu.sync_copy(dshared.at[pl.ds(r0, RPT)], dpart_hbm.at[cid, pl.ds(r0, RPT)])


_edge1 = functools.partial(
    pl.kernel,
    out_type=[jax.ShapeDtypeStruct((E,), jnp.float32),
              jax.ShapeDtypeStruct((NC, NPAD), jnp.float32)],
    mesh=plsc.VectorSubcoreMesh(core_axis_name="c", subcore_axis_name="s"),
    compiler_params=pltpu.CompilerParams(
        needs_layout_passes=False, use_tc_tiling_on_sc=False),
    scratch_types=[
        pltpu.VMEM((4, C), jnp.int32),
        pltpu.VMEM((4, C), jnp.int32),
        pltpu.SemaphoreType.DMA,
        pltpu.SemaphoreType.DMA,
        pltpu.SemaphoreType.DMA,
        pltpu.SemaphoreType.DMA,
        pltpu.VMEM((C, DW), jnp.int32),
        pltpu.VMEM((C, DW), jnp.int32),
        pltpu.SemaphoreType.DMA,
        pltpu.SemaphoreType.DMA,
        pltpu.VMEM((C, DW), jnp.int32),
        pltpu.VMEM((C, DW), jnp.int32),
        pltpu.SemaphoreType.DMA,
        pltpu.SemaphoreType.DMA,
        pltpu.VMEM((C,), jnp.float32),
        pltpu.VMEM((2 * DW,), jnp.int32),
        pltpu.VMEM((RPT,), jnp.float32),
        pltpu.VMEM_SHARED((NPAD,), jnp.float32),
    ],
)(_edge1_body)


def _edge2_body(x_hbm, src_hbm, dst_hbm, ex_hbm,
                opart_hbm,
                sidxs, didxs, exbs, is0, is1, is2, is3,
                rows0, sG0, sS0,
                rows1, sG1, sS1,
                oshared):
  cid = lax.axis_index("c")
  sid = lax.axis_index("s")
  wid = cid * NS + sid
  isems = (is0, is1, is2, is3)

  def _z(i, _):
    for r in range(D // L):
      rows0[i, pl.ds(r * L, L)] = jnp.zeros((L,), jnp.float32)
    return 0
  lax.fori_loop(0, C, _z, 0)

  z0 = sid * RPT
  for q in range(RPT // C):
    pltpu.sync_copy(rows0, oshared.at[pl.ds(z0 + q * C, C)])

  plsc.subcore_barrier()

  gsets = ((rows0, sG0, sS0), (rows1, sG1, sS1))

  def istage(k, q):
    chunk = wid + k * NW

    @pl.when(chunk < NCHUNK)
    def _():
      off = chunk * C
      pltpu.async_copy(src_hbm.at[pl.ds(off, C)], sidxs.at[q], isems[q])
      pltpu.async_copy(dst_hbm.at[pl.ds(off, C)], didxs.at[q], isems[q])
      pltpu.async_copy(ex_hbm.at[pl.ds(off, C)], exbs.at[q], isems[q])

  def gstage(k, q, p):
    rows, sG, sS = gsets[p]
    chunk = wid + k * NW
    chunk_prev = wid + (k - 2) * NW

    @pl.when((k >= 2) & (chunk_prev < NCHUNK))
    def _():
      pltpu.make_async_copy(
          rows, oshared.at[didxs.at[(q + 2) % 4]], sS).wait()

    @pl.when(chunk < NCHUNK)
    def _():
      off = chunk * C
      pltpu.make_async_copy(
          src_hbm.at[pl.ds(off, C)], sidxs.at[q], isems[q]).wait()
      pltpu.make_async_copy(
          dst_hbm.at[pl.ds(off, C)], didxs.at[q], isems[q]).wait()
      pltpu.make_async_copy(
          ex_hbm.at[pl.ds(off, C)], exbs.at[q], isems[q]).wait()
      pltpu.async_copy(x_hbm.at[sidxs.at[q]], rows, sG)

  def cstage(k, q, p):
    rows, sG, sS = gsets[p]
    chunk = wid + k * NW

    @pl.when(chunk < NCHUNK)
    def _():
      pltpu.make_async_copy(x_hbm.at[sidxs.at[q]], rows, sG).wait()
      exb = exbs.at[q]

      @plsc.parallel_loop(0, C, unroll=8)
      def _(e):
        avec = plsc.load_gather(exb, [jnp.full((L,), e, jnp.int32)])
        for r in range(D // L):
          rows[e, pl.ds(r * L, L)] = rows[e, pl.ds(r * L, L)] * avec

      pltpu.async_copy(rows, oshared.at[didxs.at[q]], sS, add=True)

  istage(0, 0)
  istage(1, 1)
  istage(2, 2)
  gstage(0, 0, 0)
  gstage(1, 1, 1)

  def loop_body(j, _):
    k0 = 4 * j
    istage(k0 + 3, 3)
    cstage(k0, 0, 0)
    gstage(k0 + 2, 2, 0)
    istage(k0 + 4, 0)
    cstage(k0 + 1, 1, 1)
    gstage(k0 + 3, 3, 1)
    istage(k0 + 5, 1)
    cstage(k0 + 2, 2, 0)
    gstage(k0 + 4, 0, 0)
    istage(k0 + 6, 2)
    cstage(k0 + 3, 3, 1)
    gstage(k0 + 5, 1, 1)
    return 0

  lax.fori_loop(0, KMAX4, loop_body, 0)
  plsc.subcore_barrier()

  for q in range(RPT // C):
    start = z0 + q * C
    pltpu.sync_copy(oshared.at[pl.ds(start, C)],
                    opart_hbm.at[cid, pl.ds(start, C)])


_edge2 = functools.partial(
    pl.kernel,
    out_type=jax.ShapeDtypeStruct((NC, NPAD, D), jnp.float32),
    mesh=plsc.VectorSubcoreMesh(core_axis_name="c", subcore_axis_name="s"),
    compiler_params=pltpu.CompilerParams(needs_layout_passes=False),
    scratch_types=[
        pltpu.VMEM((4, C), jnp.int32),
        pltpu.VMEM((4, C), jnp.int32),
        pltpu.VMEM((4, C), jnp.float32),
        pltpu.SemaphoreType.DMA,
        pltpu.SemaphoreType.DMA,
        pltpu.SemaphoreType.DMA,
        pltpu.SemaphoreType.DMA,
        pltpu.VMEM((C, D), jnp.float32),
        pltpu.SemaphoreType.DMA,
        pltpu.SemaphoreType.DMA,
        pltpu.VMEM((C, D), jnp.float32),
        pltpu.SemaphoreType.DMA,
        pltpu.SemaphoreType.DMA,
        pltpu.VMEM_SHARED((NPAD, D), jnp.float32),
    ],
)(_edge2_body)


def kernel(x, edge_index, W, b_lin, att):
  src = edge_index[0].astype(jnp.int32)
  dst = edge_index[1].astype(jnp.int32)
  h16 = _tc_h(x, W, b_lin)
  h32 = lax.bitcast_convert_type(h16.reshape(N, DW, 2), jnp.int32)
  att32 = lax.bitcast_convert_type(
      att.astype(jnp.bfloat16).reshape(DW, 2), jnp.int32)
  att32 = jnp.concatenate([att32, jnp.zeros((DW,), jnp.int32)])
  ex, dpart = _edge1(h32, src, dst, att32)
  opart = _edge2(x, src, dst, ex)
  return _tc_merge(opart, dpart)


# speedup vs baseline: 1.0724x; 1.0724x over previous
import functools

import jax
import jax.numpy as jnp
import numpy as np
from jax import lax
from jax.experimental import pallas as pl
from jax.experimental.pallas import tpu as pltpu
from jax.experimental.pallas import tpu_sc as plsc

N = 10000
E = 320000
D = 128

NC = 2
NS = 16
L = 16
NW = NC * NS
C = 128
DW = D // 2
NCHUNK = E // C
KMAX = -(-NCHUNK // NW)
KMAX4 = -(-KMAX // 4)
NPAD = 10240
RPT = NPAD // NS


def _h_body(x_ref, w_ref, b_ref, o_ref):
  h = jnp.dot(x_ref[...], w_ref[...], preferred_element_type=jnp.float32)
  o_ref[...] = (h + b_ref[...]).astype(jnp.bfloat16)


def _tc_h(x, w, b):
  return pl.pallas_call(
      _h_body,
      grid=(10,),
      in_specs=[
          pl.BlockSpec((N // 10, D), lambda i: (i, 0)),
          pl.BlockSpec((D, D), lambda i: (0, 0)),
          pl.BlockSpec((1, D), lambda i: (0, 0)),
      ],
      out_specs=pl.BlockSpec((N // 10, D), lambda i: (i, 0)),
      out_shape=jax.ShapeDtypeStruct((N, D), jnp.bfloat16),
  )(x, w, b.reshape(1, D))


def _merge_body(p_ref, d_ref, o_ref):
  dsum = d_ref[0, :] + d_ref[1, :] + 1e-16
  o_ref[...] = (p_ref[0] + p_ref[1]) * (1.0 / dsum)[:, None]


def _tc_merge(opart, dpart):
  return pl.pallas_call(
      _merge_body,
      grid=(NPAD // 1024,),
      in_specs=[
          pl.BlockSpec((2, 1024, D), lambda i: (0, i, 0)),
          pl.BlockSpec((2, 1024), lambda i: (0, i)),
      ],
      out_specs=pl.BlockSpec((1024, D), lambda i: (i, 0)),
      out_shape=jax.ShapeDtypeStruct((NPAD, D), jnp.float32),
  )(opart, dpart)[:N]


def _edge1_body(h_hbm, src_hbm, dst_hbm, att_hbm,
                ex_hbm, dpart_hbm,
                sidxs, didxs, is0, is1, is2, is3,
                hs0, hd0, sA0, sB0,
                hs1, hd1, sA1, sB1,
                lg, attv, zb, dshared):
  cid = lax.axis_index("c")
  sid = lax.axis_index("s")
  wid = cid * NS + sid
  r0 = sid * RPT
  isems = (is0, is1, is2, is3)

  def _z(i, _):
    zb[pl.ds(i * L, L)] = jnp.zeros((L,), jnp.float32)
    return 0
  lax.fori_loop(0, RPT // L, _z, 0)
  pltpu.sync_copy(zb, dshared.at[pl.ds(r0, RPT)])

  pltpu.sync_copy(att_hbm, attv)
  av = [plsc.bitcast(attv[pl.ds(r * L, L)], jnp.bfloat16)
        for r in range(DW // L)]
  plsc.subcore_barrier()

  gsets = ((hs0, hd0, sA0, sB0), (hs1, hd1, sA1, sB1))

  def istage(k, q):
    chunk = wid + k * NW

    @pl.when(chunk < NCHUNK)
    def _():
      off = chunk * C
      pltpu.async_copy(src_hbm.at[pl.ds(off, C)], sidxs.at[q], isems[q])
      pltpu.async_copy(dst_hbm.at[pl.ds(off, C)], didxs.at[q], isems[q])

  def gstage(k, q, p):
    hs, hd, sA, sB = gsets[p]
    chunk = wid + k * NW

    @pl.when(chunk < NCHUNK)
    def _():
      off = chunk * C
      pltpu.make_async_copy(
          src_hbm.at[pl.ds(off, C)], sidxs.at[q], isems[q]).wait()
      pltpu.make_async_copy(
          dst_hbm.at[pl.ds(off, C)], didxs.at[q], isems[q]).wait()
      pltpu.async_copy(h_hbm.at[sidxs.at[q]], hs, sA)
      pltpu.async_copy(h_hbm.at[didxs.at[q]], hd, sB)

  def cstage(k, q, p):
    hs, hd, sA, sB = gsets[p]
    chunk = wid + k * NW

    @pl.when(chunk < NCHUNK)
    def _():
      off = chunk * C
      pltpu.make_async_copy(h_hbm.at[sidxs.at[q]], hs, sA).wait()
      pltpu.make_async_copy(h_hbm.at[didxs.at[q]], hd, sB).wait()

      @plsc.parallel_loop(0, C, unroll=4)
      def _(e):
        acc = jnp.zeros((L,), jnp.float32)
        for r in range(DW // L):
          zs = plsc.bitcast(hs[e, pl.ds(r * L, L)], jnp.bfloat16)
          zd = plsc.bitcast(hd[e, pl.ds(r * L, L)], jnp.bfloat16)
          z = zs + zd
          t = jnp.maximum(z, z * jnp.bfloat16(0.2))
          pa, pb = plsc.unpack(t * av[r], format=plsc.PackFormat.INTERLEAVED)
          acc = acc + pa + pb
        lane = lax.iota(jnp.int32, L)
        for s in (8, 4, 2, 1):
          acc = acc + jnp.take_along_axis(
              acc, lane ^ s, axis=0, mode="promise_in_bounds")
        plsc.store_scatter(lg, [jnp.full((L,), e, jnp.int32)], acc)

      for g in range(C // L):
        v = lg[pl.ds(g * L, L)]
        lg[pl.ds(g * L, L)] = jnp.exp(jnp.minimum(v, 60.0))
      pltpu.sync_copy(lg, ex_hbm.at[pl.ds(off, C)])
      pltpu.sync_copy(lg, dshared.at[didxs.at[q]], add=True)

  istage(0, 0)
  istage(1, 1)
  istage(2, 2)
  gstage(0, 0, 0)
  gstage(1, 1, 1)

  def loop_body(j, _):
    k0 = 4 * j
    istage(k0 + 3, 3)
    cstage(k0, 0, 0)
    gstage(k0 + 2, 2, 0)
    istage(k0 + 4, 0)
    cstage(k0 + 1, 1, 1)
    gstage(k0 + 3, 3, 1)
    istage(k0 + 5, 1)
    cstage(k0 + 2, 2, 0)
    gstage(k0 + 4, 0, 0)
    istage(k0 + 6, 2)
    cstage(k0 + 3, 3, 1)
    gstage(k0 + 5, 1, 1)
    return 0

  lax.fori_loop(0, KMAX4, loop_body, 0)
  plsc.subcore_barrier()
  pltpu.sync_copy(dshared.at[pl.ds(r0, RPT)], dpart_hbm.at[cid, pl.ds(r0, RPT)])


_edge1 = functools.partial(
    pl.kernel,
    out_type=[jax.ShapeDtypeStruct((E,), jnp.float32),
              jax.ShapeDtypeStruct((NC, NPAD), jnp.float32)],
    mesh=plsc.VectorSubcoreMesh(core_axis_name="c", subcore_axis_name="s"),
    compiler_params=pltpu.CompilerParams(
        needs_layout_passes=False, use_tc_tiling_on_sc=False),
    scratch_types=[
        pltpu.VMEM((4, C), jnp.int32),
        pltpu.VMEM((4, C), jnp.int32),
        pltpu.SemaphoreType.DMA,
        pltpu.SemaphoreType.DMA,
        pltpu.SemaphoreType.DMA,
        pltpu.SemaphoreType.DMA,
        pltpu.VMEM((C, DW), jnp.int32),
        pltpu.VMEM((C, DW), jnp.int32),
        pltpu.SemaphoreType.DMA,
        pltpu.SemaphoreType.DMA,
        pltpu.VMEM((C, DW), jnp.int32),
        pltpu.VMEM((C, DW), jnp.int32),
        pltpu.SemaphoreType.DMA,
        pltpu.SemaphoreType.DMA,
        pltpu.VMEM((C,), jnp.float32),
        pltpu.VMEM((2 * DW,), jnp.int32),
        pltpu.VMEM((RPT,), jnp.float32),
        pltpu.VMEM_SHARED((NPAD,), jnp.float32),
    ],
)(_edge1_body)


def _edge2_body(x_hbm, src_hbm, dst_hbm, ex_hbm,
                opart_hbm,
                sidxs, didxs, exbs, is0, is1, is2, is3,
                rows0, sG0,
                rows1, sG1,
                stg, oshared):
  cid = lax.axis_index("c")
  sid = lax.axis_index("s")
  wid = cid * NS + sid
  isems = (is0, is1, is2, is3)

  def _z(i, _):
    for r in range(D // L):
      stg[i, pl.ds(r * L, L)] = jnp.zeros((L,), jnp.float32)
    return 0
  lax.fori_loop(0, C, _z, 0)

  z0 = sid * RPT
  for q in range(RPT // C):
    pltpu.sync_copy(stg, oshared.at[pl.ds(z0 + q * C, C)])

  plsc.subcore_barrier()

  gsets = ((rows0, sG0), (rows1, sG1))

  def istage(k, q):
    chunk = wid + k * NW

    @pl.when(chunk < NCHUNK)
    def _():
      off = chunk * C
      pltpu.async_copy(src_hbm.at[pl.ds(off, C)], sidxs.at[q], isems[q])
      pltpu.async_copy(dst_hbm.at[pl.ds(off, C)], didxs.at[q], isems[q])
      pltpu.async_copy(ex_hbm.at[pl.ds(off, C)], exbs.at[q], isems[q])

  def gstage(k, q, p):
    rows, sG = gsets[p]
    chunk = wid + k * NW

    @pl.when(chunk < NCHUNK)
    def _():
      off = chunk * C
      pltpu.make_async_copy(
          src_hbm.at[pl.ds(off, C)], sidxs.at[q], isems[q]).wait()
      pltpu.make_async_copy(
          dst_hbm.at[pl.ds(off, C)], didxs.at[q], isems[q]).wait()
      pltpu.make_async_copy(
          ex_hbm.at[pl.ds(off, C)], exbs.at[q], isems[q]).wait()
      pltpu.async_copy(x_hbm.at[sidxs.at[q]], rows, sG)

  def cstage(k, q, p):
    rows, sG = gsets[p]
    chunk = wid + k * NW

    @pl.when(chunk < NCHUNK)
    def _():
      pltpu.make_async_copy(x_hbm.at[sidxs.at[q]], rows, sG).wait()
      exb = exbs.at[q]

      @plsc.parallel_loop(0, C, unroll=4)
      def _(e):
        avec = plsc.load_gather(exb, [jnp.full((L,), e, jnp.int32)])
        av16 = plsc.pack(avec, avec, format=plsc.PackFormat.INTERLEAVED)
        for r in range(DW // L):
          v = plsc.bitcast(rows[e, pl.ds(r * L, L)], jnp.bfloat16)
          pa, pb = plsc.unpack(v * av16, format=plsc.PackFormat.INTERLEAVED)
          stg[e, pl.ds(2 * r * L, L)] = pa
          stg[e, pl.ds((2 * r + 1) * L, L)] = pb

      pltpu.sync_copy(stg, oshared.at[didxs.at[q]], add=True)

  istage(0, 0)
  istage(1, 1)
  istage(2, 2)
  gstage(0, 0, 0)
  gstage(1, 1, 1)

  def loop_body(j, _):
    k0 = 4 * j
    istage(k0 + 3, 3)
    cstage(k0, 0, 0)
    gstage(k0 + 2, 2, 0)
    istage(k0 + 4, 0)
    cstage(k0 + 1, 1, 1)
    gstage(k0 + 3, 3, 1)
    istage(k0 + 5, 1)
    cstage(k0 + 2, 2, 0)
    gstage(k0 + 4, 0, 0)
    istage(k0 + 6, 2)
    cstage(k0 + 3, 3, 1)
    gstage(k0 + 5, 1, 1)
    return 0

  lax.fori_loop(0, KMAX4, loop_body, 0)
  plsc.subcore_barrier()

  for q in range(RPT // C):
    start = z0 + q * C
    pltpu.sync_copy(oshared.at[pl.ds(start, C)],
                    opart_hbm.at[cid, pl.ds(start, C)])


_edge2 = functools.partial(
    pl.kernel,
    out_type=jax.ShapeDtypeStruct((NC, NPAD, D), jnp.float32),
    mesh=plsc.VectorSubcoreMesh(core_axis_name="c", subcore_axis_name="s"),
    compiler_params=pltpu.CompilerParams(
        needs_layout_passes=False, use_tc_tiling_on_sc=False),
    scratch_types=[
        pltpu.VMEM((4, C), jnp.int32),
        pltpu.VMEM((4, C), jnp.int32),
        pltpu.VMEM((4, C), jnp.float32),
        pltpu.SemaphoreType.DMA,
        pltpu.SemaphoreType.DMA,
        pltpu.SemaphoreType.DMA,
        pltpu.SemaphoreType.DMA,
        pltpu.VMEM((C, DW), jnp.int32),
        pltpu.SemaphoreType.DMA,
        pltpu.VMEM((C, DW), jnp.int32),
        pltpu.SemaphoreType.DMA,
        pltpu.VMEM((C, D), jnp.float32),
        pltpu.VMEM_SHARED((NPAD, D), jnp.float32),
    ],
)(_edge2_body)


_XPERM = np.empty((D,), np.int32)
for _g in range(D // 32):
  for _i in range(16):
    _XPERM[32 * _g + 2 * _i] = 32 * _g + _i
    _XPERM[32 * _g + 2 * _i + 1] = 32 * _g + 16 + _i


def kernel(x, edge_index, W, b_lin, att):
  src = edge_index[0].astype(jnp.int32)
  dst = edge_index[1].astype(jnp.int32)
  h16 = _tc_h(x, W, b_lin)
  h32 = lax.bitcast_convert_type(h16.reshape(N, DW, 2), jnp.int32)
  att32 = lax.bitcast_convert_type(
      att.astype(jnp.bfloat16).reshape(DW, 2), jnp.int32)
  att32 = jnp.concatenate([att32, jnp.zeros((DW,), jnp.int32)])
  x16p = x[:, _XPERM].astype(jnp.bfloat16)
  x32 = lax.bitcast_convert_type(x16p.reshape(N, DW, 2), jnp.int32)
  ex, dpart = _edge1(h32, src, dst, att32)
  opart = _edge2(x32, src, dst, ex)
  return _tc_merge(opart, dpart)
